# baseline (speedup 1.0000x reference)
import functools

import jax
import jax.numpy as jnp
from jax import lax
from jax.experimental import pallas as pl
from jax.experimental.pallas import tpu as pltpu
from jax.experimental.pallas import tpu_sc as plsc

_NC = 2
_NS = 16
_NW = _NC * _NS


def _lrelu(x):
    return jnp.where(x >= 0, x, 0.2 * x)


def _dot(a, b):
    return jnp.dot(a, b, preferred_element_type=jnp.float32)


def _sc_gather(table, gidx4d, d):
    nw, outer, inner, lanes = gidx4d.shape
    assert outer % 2 == 0
    ch = inner * lanes
    out_rows = nw * outer * ch
    mesh = plsc.VectorSubcoreMesh(core_axis_name="c", subcore_axis_name="s")

    @functools.partial(
        pl.kernel,
        out_type=jax.ShapeDtypeStruct((out_rows, d), jnp.float32),
        mesh=mesh,
        scratch_types=[
            pltpu.VMEM((inner, lanes), jnp.int32),
            pltpu.VMEM((inner, lanes), jnp.int32),
            pltpu.VMEM((ch, d), jnp.float32),
            pltpu.VMEM((ch, d), jnp.float32),
            pltpu.SemaphoreType.DMA,
            pltpu.SemaphoreType.DMA,
            pltpu.SemaphoreType.DMA,
            pltpu.SemaphoreType.DMA,
        ],
        compiler_params=pltpu.CompilerParams(use_tc_tiling_on_sc=False),
    )
    def gk(table_hbm, gidx_hbm, out_hbm, idx0, idx1, rows0, rows1,
           semg0, semg1, semw0, semw1):
        wid = lax.axis_index("s") * _NC + lax.axis_index("c")
        idx_v = (idx0, idx1)
        rows_v = (rows0, rows1)
        semg = (semg0, semg1)
        semw = (semw0, semw1)

        def body(u, carry):
            t0 = u * 2
            cps = {}
            for b in (0, 1):
                t = t0 + b

                @pl.when(t >= 2)
                def _():
                    pltpu.make_async_copy(
                        rows_v[b], out_hbm.at[pl.ds(0, ch)], semw[b]
                    ).wait()

                pltpu.sync_copy(gidx_hbm.at[wid, t], idx_v[b])
                cps[b] = [
                    pltpu.async_copy(
                        table_hbm.at[idx_v[b].at[j]],
                        rows_v[b].at[pl.ds(j * lanes, lanes)],
                        semg[b],
                    )
                    for j in range(inner)
                ]
            for b in (0, 1):
                t = t0 + b
                for cp in cps[b]:
                    cp.wait()
                base = (wid * outer + t) * ch
                pltpu.async_copy(rows_v[b], out_hbm.at[pl.ds(base, ch)],
                                 semw[b])
            return carry

        lax.fori_loop(0, outer // 2, body, 0)
        for b in (0, 1):
            pltpu.make_async_copy(
                rows_v[b], out_hbm.at[pl.ds(0, ch)], semw[b]
            ).wait()

    return gk(table, gidx4d)


def _pad_reshape_idx(gidx_flat, outer, inner):
    total = _NW * outer * inner * 128
    pad = total - gidx_flat.shape[0]
    gp = jnp.pad(gidx_flat, (0, pad))
    return gp.reshape(_NW, outer, inner, 128)


def _wspec(shp):
    return pl.BlockSpec(shp, lambda i: tuple(0 for _ in shp))


def _stage_c(g1r, tbl, consts, rb):
    bn_pad = tbl.shape[0]
    nblk = bn_pad // rb
    (pctr, p2, bd1, bd2, bd3, gk_, g3, m1, wscp, w0l, b1l, bfc1l, b3l,
     bml1, bsc, mask3) = consts

    def body(g1_ref, tbl_ref, pctr_r, p2_r, bd1_r, bd2_r, bd3_r, g_r, g3_r,
             m1_r, wscp_r, w0l_r, b1l_r, bfc1l_r, b3l_r, bml1_r, bsc_r,
             mask3_r, agg_ref, fx2_ref, sc_ref):
        x = g1_ref[...]
        t = tbl_ref[...]
        center = _dot(t, pctr_r[...])
        relm = (center - x) * mask3_r[...]
        dist2 = _dot(relm * relm, g3_r[...])
        dist = jnp.sqrt(dist2 + 1e-12)
        pre = (_dot(x, bd1_r[...]) + dist * w0l_r[...] + _dot(t, p2_r[...])
               + b1l_r[...])
        fcat = _lrelu(pre)
        att = _dot(fcat, bd2_r[...]) + bfc1l_r[...]
        m = att
        for sh in (16, 32, 64, 128):
            m = jnp.maximum(m, pltpu.roll(m, sh, 1))
        e = jnp.exp(att - m)
        den = _dot(e, g_r[...])
        num = _dot(e * fcat, g_r[...])
        aggf = num * (1.0 / den)
        agg_ref[...] = _lrelu(_dot(aggf, m1_r[...]) + bml1_r[...])
        fx2_ref[...] = _lrelu(_dot(fcat, bd3_r[...]) + b3l_r[...])
        sc_ref[...] = _dot(t, wscp_r[...]) + bsc_r[...]

    return pl.pallas_call(
        body,
        grid=(nblk,),
        in_specs=[
            pl.BlockSpec((rb, 256), lambda i: (i, 0)),
            pl.BlockSpec((rb, 16), lambda i: (i, 0)),
        ] + [_wspec(c.shape) for c in consts],
        out_specs=[
            pl.BlockSpec((rb, 8), lambda i: (i, 0)),
            pl.BlockSpec((rb, 128), lambda i: (i, 0)),
            pl.BlockSpec((rb, 32), lambda i: (i, 0)),
        ],
        out_shape=[
            jax.ShapeDtypeStruct((bn_pad, 8), jnp.float32),
            jax.ShapeDtypeStruct((bn_pad, 128), jnp.float32),
            jax.ShapeDtypeStruct((bn_pad, 32), jnp.float32),
        ],
    )(g1r, tbl, *consts)


def _stage_e(g2r, fx2r, scv, consts, rb):
    bn_pad = scv.shape[0]
    nblk = bn_pad // rb
    (e1, e2, bd2e, gk_, m2, wm2, bfc2l, bml2, bm2) = consts

    def body(g2_ref, fx2_ref, sc_ref, e1_r, e2_r, bd2e_r, g_r, m2_r, wm2_r,
             bfc2l_r, bml2_r, bm2_r, enc_ref):
        fcat = _dot(g2_ref[...], e1_r[...]) + _dot(fx2_ref[...], e2_r[...])
        att = _dot(fcat, bd2e_r[...]) + bfc2l_r[...]
        m = att
        for sh in (16, 32, 64, 128):
            m = jnp.maximum(m, pltpu.roll(m, sh, 1))
        e = jnp.exp(att - m)
        den = _dot(e, g_r[...])
        num = _dot(e * fcat, g_r[...])
        aggf = num * (1.0 / den)
        f_lfa = _lrelu(_dot(aggf, m2_r[...]) + bml2_r[...])
        f_main = _dot(f_lfa, wm2_r[...]) + bm2_r[...]
        enc_ref[...] = _lrelu(f_main + sc_ref[...])

    return pl.pallas_call(
        body,
        grid=(nblk,),
        in_specs=[
            pl.BlockSpec((rb, 128), lambda i: (i, 0)),
            pl.BlockSpec((rb, 128), lambda i: (i, 0)),
            pl.BlockSpec((rb, 32), lambda i: (i, 0)),
        ] + [_wspec(c.shape) for c in consts],
        out_specs=[pl.BlockSpec((rb, 32), lambda i: (i, 0))],
        out_shape=[jax.ShapeDtypeStruct((bn_pad, 32), jnp.float32)],
    )(g2r, fx2r, scv, *consts)[0]


def _stage_g(g3, rb, k):
    rows_pad = g3.shape[0]
    nblk = rows_pad // (rb * k)

    def body(g_ref, out_ref):
        out_ref[...] = jnp.max(g_ref[...].reshape(rb, k, 32), axis=1)

    return pl.pallas_call(
        body,
        grid=(nblk,),
        in_specs=[pl.BlockSpec((rb * k, 32), lambda i: (i, 0))],
        out_specs=[pl.BlockSpec((rb, 32), lambda i: (i, 0))],
        out_shape=[jax.ShapeDtypeStruct((rows_pad // k, 32), jnp.float32)],
    )(g3)[0]


def _stage_i(g4, Wfc, bfc, rows, rb):
    nblk = rows // rb

    def body(g_ref, wfc, rbfc, out_ref):
        out_ref[...] = _dot(g_ref[...], wfc[...]) + rbfc[...]

    return pl.pallas_call(
        body,
        grid=(nblk,),
        in_specs=[
            pl.BlockSpec((rb, 32), lambda i: (i, 0)),
            _wspec(Wfc.shape), _wspec(bfc.shape),
        ],
        out_specs=[pl.BlockSpec((rb, 13), lambda i: (i, 0))],
        out_shape=[jax.ShapeDtypeStruct((rows, 13), jnp.float32)],
    )(g4, Wfc, bfc)[0]


def kernel(xyz, features, neigh_idx, sub_idx, interp_idx, W_mlp1, b_mlp1,
           W_lfa1, b_lfa1, W_att1_fc, b_att1_fc, W_att1_mlp, b_att1_mlp,
           W_lfa2, b_lfa2, W_att2_fc, b_att2_fc, W_att2_mlp, b_att2_mlp,
           W_mlp2, b_mlp2, W_sc, b_sc, W_fc, b_fc):
    B, N, K = neigh_idx.shape
    Ns = sub_idx.shape[1]
    BN = B * N
    RB = 512
    BN_PAD = 200704
    f32 = jnp.float32
    offs_n = (jnp.arange(B, dtype=jnp.int32) * N)[:, None, None]
    offs_s = (jnp.arange(B, dtype=jnp.int32) * Ns)[:, None]
    eyeK = jnp.eye(K, dtype=f32)
    onesK = jnp.ones((K, K), f32)

    pctr = jnp.tile(jnp.eye(16, dtype=f32), (1, K)) * (
        (jnp.arange(16) < 3).astype(f32)[:, None])
    wc = W_lfa1[1:4] + W_lfa1[4:7]
    p2 = jnp.zeros((16, 16), f32).at[0:3, 8:16].set(wc)
    p2 = jnp.tile(p2, (1, K))
    blk1 = (jnp.zeros((16, 16), f32)
            .at[0:3, 8:16].set(W_lfa1[7:10] - W_lfa1[1:4])
            .at[3:11, 0:8].set(W_mlp1))
    bd1 = jnp.kron(eyeK, blk1)
    bd2 = jnp.kron(eyeK, W_att1_fc)
    bd3 = jnp.kron(eyeK, jnp.zeros((16, 8), f32).at[8:16].set(W_lfa2))
    gk_ = jnp.kron(onesK, jnp.eye(16, dtype=f32))
    g3 = jnp.kron(eyeK, jnp.ones((16, 16), f32))
    m1 = jnp.zeros((256, 8), f32).at[0:16].set(W_att1_mlp)
    wscp = jnp.zeros((16, 32), f32).at[3:11].set(W_sc)
    w0l = jnp.tile(jnp.zeros((16,), f32).at[8:16].set(W_lfa1[0]), K)[None]
    b1l = jnp.tile(jnp.concatenate([b_mlp1, b_lfa1]), K)[None]
    bfc1l = jnp.tile(b_att1_fc, K)[None]
    b3l = jnp.tile(b_lfa2, K)[None]
    mask3 = ((jnp.arange(256) % 16) < 3).astype(f32)[None]
    consts_c = (pctr, p2, bd1, bd2, bd3, gk_, g3, m1, wscp, w0l, b1l,
                bfc1l, b3l, b_att1_mlp[None], b_sc[None], mask3)

    e1 = jnp.kron(eyeK, jnp.concatenate(
        [jnp.eye(8, dtype=f32), jnp.zeros((8, 8), f32)], axis=1))
    e2 = jnp.kron(eyeK, jnp.concatenate(
        [jnp.zeros((8, 8), f32), jnp.eye(8, dtype=f32)], axis=1))
    bd2e = jnp.kron(eyeK, W_att2_fc)
    m2 = jnp.zeros((256, 16), f32).at[0:16].set(W_att2_mlp)
    bfc2l = jnp.tile(b_att2_fc, K)[None]
    consts_e = (e1, e2, bd2e, gk_, m2, W_mlp2, bfc2l, b_att2_mlp[None],
                b_mlp2[None])

    table1 = jnp.concatenate(
        [xyz, features, jnp.zeros((B, N, 5), f32)], axis=-1
    ).reshape(BN, 16)

    gidx1 = _pad_reshape_idx((neigh_idx + offs_n).reshape(-1), 56, 14)
    g1 = _sc_gather(table1, gidx1, 16)

    tbl_pad = jnp.pad(table1, ((0, BN_PAD - BN), (0, 0)))
    f_pc_agg, f_xyz2, sc_v = _stage_c(
        g1.reshape(BN_PAD, 256), tbl_pad, consts_c, RB)

    g2 = _sc_gather(f_pc_agg, gidx1, 8)

    f_enc = _stage_e(g2.reshape(BN_PAD, 128), f_xyz2, sc_v, consts_e, RB)

    gidx3 = _pad_reshape_idx((sub_idx + offs_n).reshape(-1), 14, 14)
    g3g = _sc_gather(f_enc, gidx3, 32)
    f_sampled = _stage_g(g3g, RB, K)

    gidx4 = _pad_reshape_idx((interp_idx[:, :, 0] + offs_s).reshape(-1), 4, 14)
    g4 = _sc_gather(f_sampled, gidx4, 32)

    logits = _stage_i(g4, W_fc, b_fc[None], BN, 2000)
    return logits.reshape(B, N, 13)


# speedup vs baseline: 1.0698x; 1.0698x over previous
import functools

import jax
import jax.numpy as jnp
from jax import lax
from jax.experimental import pallas as pl
from jax.experimental.pallas import tpu as pltpu
from jax.experimental.pallas import tpu_sc as plsc

_NC = 2
_NS = 16
_NW = _NC * _NS


def _lrelu(x):
    return jnp.where(x >= 0, x, 0.2 * x)


def _dot(a, b):
    return jnp.dot(a, b, preferred_element_type=jnp.float32)


def _sc_gather(table, gidx4d, d, out_dtype=jnp.float32):
    nw, outer, inner, lanes = gidx4d.shape
    ch = inner * lanes
    out_rows = nw * outer * ch
    mesh = plsc.VectorSubcoreMesh(core_axis_name="c", subcore_axis_name="s")

    @functools.partial(
        pl.kernel,
        out_type=jax.ShapeDtypeStruct((out_rows, d), out_dtype),
        mesh=mesh,
        scratch_types=[
            pltpu.VMEM((inner, lanes), jnp.int32),
            pltpu.VMEM((ch, d), out_dtype),
            pltpu.SemaphoreType.DMA,
        ],
        compiler_params=pltpu.CompilerParams(use_tc_tiling_on_sc=False),
    )
    def gk(table_hbm, gidx_hbm, out_hbm, idx_v, rows_v, sem):
        wid = lax.axis_index("s") * _NC + lax.axis_index("c")

        def body(t, carry):
            pltpu.sync_copy(gidx_hbm.at[wid, t], idx_v)
            cps = []
            for j in range(inner):
                cps.append(
                    pltpu.async_copy(
                        table_hbm.at[idx_v.at[j]],
                        rows_v.at[pl.ds(j * lanes, lanes)],
                        sem,
                    )
                )
            for cp in cps:
                cp.wait()
            base = (wid * outer + t) * ch
            pltpu.sync_copy(rows_v, out_hbm.at[pl.ds(base, ch)])
            return carry

        lax.fori_loop(0, outer, body, 0)

    return gk(table, gidx4d)


def _pad_reshape_idx(gidx_flat, outer, inner):
    total = _NW * outer * inner * 128
    pad = total - gidx_flat.shape[0]
    gp = jnp.pad(gidx_flat, (0, pad))
    return gp.reshape(_NW, outer, inner, 128)


def _wspec(shp):
    return pl.BlockSpec(shp, lambda i: tuple(0 for _ in shp))


def _stage_c(g1r, tbl, consts, rb):
    bn_pad = tbl.shape[0]
    nblk = bn_pad // rb
    bf16 = jnp.bfloat16

    def body(g1_ref, tbl_ref, p2_r, bd1_r, bd2_r, bd3_r, g_r, g3_r,
             m1_r, wscp_r, w0l_r, b1l_r, bfc1l_r, b3l_r, bml1_r, bsc_r,
             mask3_r, agg_ref, fx2_ref, sc_ref):
        x = g1_ref[...]
        t = tbl_ref[...]
        tb = t.astype(bf16)
        center = jnp.tile(t, (1, 16))
        relm = (center - x) * mask3_r[...]
        d2 = relm * relm
        dist2 = _dot(d2.astype(bf16), g3_r[...])
        dist = jnp.sqrt(dist2 + 1e-12)
        pre = (_dot(x.astype(bf16), bd1_r[...]) + dist * w0l_r[...]
               + _dot(tb, p2_r[...]) + b1l_r[...])
        fcat = _lrelu(pre)
        fcb = fcat.astype(bf16)
        att = _dot(fcb, bd2_r[...]) + bfc1l_r[...]
        m = att
        for sh in (16, 32, 64, 128):
            m = jnp.maximum(m, pltpu.roll(m, sh, 1))
        e = jnp.exp(att - m)
        den = _dot(e.astype(bf16), g_r[...])
        num = _dot((e * fcat).astype(bf16), g_r[...])
        aggf = num * (1.0 / den)
        agg_ref[...] = _lrelu(_dot(aggf.astype(bf16), m1_r[...]) + bml1_r[...])
        fx2_ref[...] = _lrelu(_dot(fcb, bd3_r[...]) + b3l_r[...]).astype(bf16)
        sc_ref[...] = (_dot(tb, wscp_r[...]) + bsc_r[...]).astype(bf16)

    return pl.pallas_call(
        body,
        grid=(nblk,),
        in_specs=[
            pl.BlockSpec((rb, 256), lambda i: (i, 0)),
            pl.BlockSpec((rb, 16), lambda i: (i, 0)),
        ] + [_wspec(c.shape) for c in consts],
        out_specs=[
            pl.BlockSpec((rb, 8), lambda i: (i, 0)),
            pl.BlockSpec((rb, 128), lambda i: (i, 0)),
            pl.BlockSpec((rb, 32), lambda i: (i, 0)),
        ],
        out_shape=[
            jax.ShapeDtypeStruct((bn_pad, 8), jnp.float32),
            jax.ShapeDtypeStruct((bn_pad, 128), jnp.bfloat16),
            jax.ShapeDtypeStruct((bn_pad, 32), jnp.bfloat16),
        ],
    )(g1r, tbl, *consts)


def _stage_e(g2r, fx2r, scv, consts, rb):
    bn_pad = scv.shape[0]
    nblk = bn_pad // rb
    bf16 = jnp.bfloat16

    def body(g2_ref, fx2_ref, sc_ref, e1_r, e2_r, bd2e_r, g_r, m2_r, wm2_r,
             bfc2l_r, bml2_r, bm2_r, enc_ref):
        fcat = (_dot(g2_ref[...].astype(bf16), e1_r[...])
                + _dot(fx2_ref[...], e2_r[...]))
        fcb = fcat.astype(bf16)
        att = _dot(fcb, bd2e_r[...]) + bfc2l_r[...]
        m = att
        for sh in (16, 32, 64, 128):
            m = jnp.maximum(m, pltpu.roll(m, sh, 1))
        e = jnp.exp(att - m)
        den = _dot(e.astype(bf16), g_r[...])
        num = _dot((e * fcat).astype(bf16), g_r[...])
        aggf = num * (1.0 / den)
        f_lfa = _lrelu(_dot(aggf.astype(bf16), m2_r[...]) + bml2_r[...])
        f_main = _dot(f_lfa.astype(bf16), wm2_r[...]) + bm2_r[...]
        enc_ref[...] = _lrelu(f_main + sc_ref[...].astype(jnp.float32))

    return pl.pallas_call(
        body,
        grid=(nblk,),
        in_specs=[
            pl.BlockSpec((rb, 128), lambda i: (i, 0)),
            pl.BlockSpec((rb, 128), lambda i: (i, 0)),
            pl.BlockSpec((rb, 32), lambda i: (i, 0)),
        ] + [_wspec(c.shape) for c in consts],
        out_specs=[pl.BlockSpec((rb, 32), lambda i: (i, 0))],
        out_shape=[jax.ShapeDtypeStruct((bn_pad, 32), jnp.float32)],
    )(g2r, fx2r, scv, *consts)[0]


def _stage_g(g3, wfcp, bfcp, rb, k):
    rows_pad = g3.shape[0]
    nblk = rows_pad // (rb * k)

    def body(g_ref, wfc_r, bfc_r, out_ref):
        mp = jnp.max(g_ref[...].reshape(rb, k, 32), axis=1)
        out_ref[...] = _dot(mp.astype(jnp.bfloat16), wfc_r[...]) + bfc_r[...]

    return pl.pallas_call(
        body,
        grid=(nblk,),
        in_specs=[pl.BlockSpec((rb * k, 32), lambda i: (i, 0)),
                  _wspec(wfcp.shape), _wspec(bfcp.shape)],
        out_specs=[pl.BlockSpec((rb, 16), lambda i: (i, 0))],
        out_shape=[jax.ShapeDtypeStruct((rows_pad // k, 16), jnp.float32)],
    )(g3, wfcp, bfcp)[0]


def _stage_i(g4, rows, rb):
    nblk = rows // rb

    def body(g_ref, out_ref):
        out_ref[...] = g_ref[:, 0:13]

    return pl.pallas_call(
        body,
        grid=(nblk,),
        in_specs=[pl.BlockSpec((rb, 16), lambda i: (i, 0))],
        out_specs=[pl.BlockSpec((rb, 13), lambda i: (i, 0))],
        out_shape=[jax.ShapeDtypeStruct((rows, 13), jnp.float32)],
    )(g4)[0]


def kernel(xyz, features, neigh_idx, sub_idx, interp_idx, W_mlp1, b_mlp1,
           W_lfa1, b_lfa1, W_att1_fc, b_att1_fc, W_att1_mlp, b_att1_mlp,
           W_lfa2, b_lfa2, W_att2_fc, b_att2_fc, W_att2_mlp, b_att2_mlp,
           W_mlp2, b_mlp2, W_sc, b_sc, W_fc, b_fc):
    B, N, K = neigh_idx.shape
    Ns = sub_idx.shape[1]
    BN = B * N
    RB = 2048
    BN_PAD = 200704
    f32 = jnp.float32
    bf16 = jnp.bfloat16
    offs_n = (jnp.arange(B, dtype=jnp.int32) * N)[:, None, None]
    offs_s = (jnp.arange(B, dtype=jnp.int32) * Ns)[:, None]
    eyeK = jnp.eye(K, dtype=f32)
    onesK = jnp.ones((K, K), f32)

    wc = W_lfa1[1:4] + W_lfa1[4:7]
    p2 = jnp.zeros((16, 16), f32).at[0:3, 8:16].set(wc)
    p2 = jnp.tile(p2, (1, K))
    blk1 = (jnp.zeros((16, 16), f32)
            .at[0:3, 8:16].set(W_lfa1[7:10] - W_lfa1[1:4])
            .at[3:11, 0:8].set(W_mlp1))
    bd1 = jnp.kron(eyeK, blk1)
    bd2 = jnp.kron(eyeK, W_att1_fc)
    bd3 = jnp.kron(eyeK, jnp.zeros((16, 8), f32).at[8:16].set(W_lfa2))
    gk_ = jnp.kron(onesK, jnp.eye(16, dtype=f32))
    g3 = jnp.kron(eyeK, jnp.ones((16, 16), f32))
    m1 = jnp.zeros((256, 8), f32).at[0:16].set(W_att1_mlp)
    wscp = jnp.zeros((16, 32), f32).at[3:11].set(W_sc)
    w0l = jnp.tile(jnp.zeros((16,), f32).at[8:16].set(W_lfa1[0]), K)[None]
    b1l = jnp.tile(jnp.concatenate([b_mlp1, b_lfa1]), K)[None]
    bfc1l = jnp.tile(b_att1_fc, K)[None]
    b3l = jnp.tile(b_lfa2, K)[None]
    mask3 = ((jnp.arange(256) % 16) < 3).astype(f32)[None]
    cast = lambda a: a.astype(bf16)
    consts_c = (cast(p2), cast(bd1), cast(bd2), cast(bd3), cast(gk_),
                cast(g3), cast(m1), cast(wscp), w0l, b1l, bfc1l, b3l,
                b_att1_mlp[None], b_sc[None], mask3)

    e1 = jnp.kron(eyeK, jnp.concatenate(
        [jnp.eye(8, dtype=f32), jnp.zeros((8, 8), f32)], axis=1))
    e2 = jnp.kron(eyeK, jnp.concatenate(
        [jnp.zeros((8, 8), f32), jnp.eye(8, dtype=f32)], axis=1))
    bd2e = jnp.kron(eyeK, W_att2_fc)
    m2 = jnp.zeros((256, 16), f32).at[0:16].set(W_att2_mlp)
    bfc2l = jnp.tile(b_att2_fc, K)[None]
    consts_e = (cast(e1), cast(e2), cast(bd2e), cast(gk_), cast(m2),
                cast(W_mlp2), bfc2l, b_att2_mlp[None], b_mlp2[None])

    wfcp = jnp.zeros((32, 16), f32).at[:, 0:13].set(W_fc)
    bfcp = jnp.zeros((1, 16), f32).at[0, 0:13].set(b_fc)

    table1 = jnp.concatenate(
        [xyz, features, jnp.zeros((B, N, 5), f32)], axis=-1
    ).reshape(BN, 16)

    gidx1 = _pad_reshape_idx((neigh_idx + offs_n).reshape(-1), 56, 14)
    g1 = _sc_gather(table1, gidx1, 16)

    tbl_pad = jnp.pad(table1, ((0, BN_PAD - BN), (0, 0)))
    f_pc_agg, f_xyz2, sc_v = _stage_c(
        g1.reshape(BN_PAD, 256), tbl_pad, consts_c, RB)

    g2 = _sc_gather(f_pc_agg, gidx1, 8)

    f_enc = _stage_e(g2.reshape(BN_PAD, 128), f_xyz2, sc_v, consts_e, RB)

    gidx3 = _pad_reshape_idx((sub_idx + offs_n).reshape(-1), 14, 14)
    g3g = _sc_gather(f_enc, gidx3, 32)
    ls = _stage_g(g3g, cast(wfcp), bfcp, 1024, K)

    gidx4 = _pad_reshape_idx((interp_idx[:, :, 0] + offs_s).reshape(-1), 4, 14)
    g4 = _sc_gather(ls, gidx4, 16)

    logits = _stage_i(g4, BN, 2000)
    return logits.reshape(B, N, 13)


# speedup vs baseline: 1.2124x; 1.1334x over previous
import functools

import jax
import jax.numpy as jnp
from jax import lax
from jax.experimental import pallas as pl
from jax.experimental.pallas import tpu as pltpu
from jax.experimental.pallas import tpu_sc as plsc

_NC = 2
_NS = 16
_NW = _NC * _NS


def _lrelu(x):
    return jnp.where(x >= 0, x, 0.2 * x)


def _dot(a, b):
    return jnp.dot(a, b, preferred_element_type=jnp.float32)


def _sc_gather(table, gidx4d, d, out_dtype=jnp.float32):
    nw, outer, inner, lanes = gidx4d.shape
    ch = inner * lanes
    out_rows = nw * outer * ch
    mesh = plsc.VectorSubcoreMesh(core_axis_name="c", subcore_axis_name="s")

    @functools.partial(
        pl.kernel,
        out_type=jax.ShapeDtypeStruct((out_rows, d), out_dtype),
        mesh=mesh,
        scratch_types=[
            pltpu.VMEM((inner, lanes), jnp.int32),
            pltpu.VMEM((ch, d), out_dtype),
            pltpu.SemaphoreType.DMA,
        ],
        compiler_params=pltpu.CompilerParams(use_tc_tiling_on_sc=False),
    )
    def gk(table_hbm, gidx_hbm, out_hbm, idx_v, rows_v, sem):
        wid = lax.axis_index("s") * _NC + lax.axis_index("c")

        def body(t, carry):
            pltpu.sync_copy(gidx_hbm.at[wid, t], idx_v)
            cps = []
            for j in range(inner):
                cps.append(
                    pltpu.async_copy(
                        table_hbm.at[idx_v.at[j]],
                        rows_v.at[pl.ds(j * lanes, lanes)],
                        sem,
                    )
                )
            for cp in cps:
                cp.wait()
            base = (wid * outer + t) * ch
            pltpu.sync_copy(rows_v, out_hbm.at[pl.ds(base, ch)])
            return carry

        lax.fori_loop(0, outer, body, 0)

    return gk(table, gidx4d)


def _pad_reshape_idx(gidx_flat, outer, inner, table_rows):
    total = _NW * outer * inner * 128
    pad = total - gidx_flat.shape[0]
    filler = jnp.arange(pad, dtype=jnp.int32) % table_rows
    gp = jnp.concatenate([gidx_flat, filler])
    return gp.reshape(_NW, outer, inner, 128)


def _wspec(shp):
    return pl.BlockSpec(shp, lambda i: tuple(0 for _ in shp))


def _stage_c(g1r, tbl, consts, rb, bn_pad):
    nblk = bn_pad // rb
    bf16 = jnp.bfloat16

    def body(g1_ref, tbl_ref, p2_r, bd1_r, bd2_r, bd3_r, g_r, g3_r,
             m1_r, wscp_r, w0l_r, b1l_r, bfc1l_r, b3l_r, bml1_r, bsc_r,
             mask3_r, agg_ref, fx2_ref, sc_ref):
        x = g1_ref[...]
        t = tbl_ref[...]
        tb = t.astype(bf16)
        center = jnp.tile(t, (1, 16))
        relm = (center - x) * mask3_r[...]
        d2 = relm * relm
        dist2 = _dot(d2.astype(bf16), g3_r[...])
        dist = jnp.sqrt(dist2 + 1e-12)
        pre = (_dot(x.astype(bf16), bd1_r[...]) + dist * w0l_r[...]
               + _dot(tb, p2_r[...]) + b1l_r[...])
        fcat = _lrelu(pre)
        fcb = fcat.astype(bf16)
        att = _dot(fcb, bd2_r[...]) + bfc1l_r[...]
        m = att
        for sh in (16, 32, 64, 128):
            m = jnp.maximum(m, pltpu.roll(m, sh, 1))
        e = jnp.exp(att - m)
        den = _dot(e.astype(bf16), g_r[...])
        num = _dot((e * fcat).astype(bf16), g_r[...])
        aggf = num * (1.0 / den)
        agg_ref[...] = _lrelu(_dot(aggf.astype(bf16), m1_r[...]) + bml1_r[...])
        fx2_ref[...] = _lrelu(_dot(fcb, bd3_r[...]) + b3l_r[...]).astype(bf16)
        sc_ref[...] = (_dot(tb, wscp_r[...]) + bsc_r[...]).astype(bf16)

    return pl.pallas_call(
        body,
        grid=(nblk,),
        in_specs=[
            pl.BlockSpec((rb, 256), lambda i: (i, 0)),
            pl.BlockSpec((rb, 16), lambda i: (i, 0)),
        ] + [_wspec(c.shape) for c in consts],
        out_specs=[
            pl.BlockSpec((rb, 8), lambda i: (i, 0)),
            pl.BlockSpec((rb, 128), lambda i: (i, 0)),
            pl.BlockSpec((rb, 32), lambda i: (i, 0)),
        ],
        out_shape=[
            jax.ShapeDtypeStruct((bn_pad, 8), jnp.float32),
            jax.ShapeDtypeStruct((bn_pad, 128), jnp.bfloat16),
            jax.ShapeDtypeStruct((bn_pad, 32), jnp.bfloat16),
        ],
    )(g1r, tbl, *consts)


def _stage_e(g2r, fx2r, scv, consts, rb):
    bn_pad = scv.shape[0]
    nblk = bn_pad // rb
    bf16 = jnp.bfloat16

    def body(g2_ref, fx2_ref, sc_ref, e1_r, e2_r, bd2e_r, g_r, m2_r, wm2_r,
             bfc2l_r, bml2_r, bm2_r, enc_ref):
        fcat = (_dot(g2_ref[...].astype(bf16), e1_r[...])
                + _dot(fx2_ref[...], e2_r[...]))
        fcb = fcat.astype(bf16)
        att = _dot(fcb, bd2e_r[...]) + bfc2l_r[...]
        m = att
        for sh in (16, 32, 64, 128):
            m = jnp.maximum(m, pltpu.roll(m, sh, 1))
        e = jnp.exp(att - m)
        den = _dot(e.astype(bf16), g_r[...])
        num = _dot((e * fcat).astype(bf16), g_r[...])
        aggf = num * (1.0 / den)
        f_lfa = _lrelu(_dot(aggf.astype(bf16), m2_r[...]) + bml2_r[...])
        f_main = _dot(f_lfa.astype(bf16), wm2_r[...]) + bm2_r[...]
        enc_ref[...] = _lrelu(f_main + sc_ref[...].astype(jnp.float32))

    return pl.pallas_call(
        body,
        grid=(nblk,),
        in_specs=[
            pl.BlockSpec((rb, 128), lambda i: (i, 0)),
            pl.BlockSpec((rb, 128), lambda i: (i, 0)),
            pl.BlockSpec((rb, 32), lambda i: (i, 0)),
        ] + [_wspec(c.shape) for c in consts],
        out_specs=[pl.BlockSpec((rb, 32), lambda i: (i, 0))],
        out_shape=[jax.ShapeDtypeStruct((bn_pad, 32), jnp.float32)],
    )(g2r, fx2r, scv, *consts)[0]


def _stage_g(g3, wfcp, bfcp, rb, k):
    rows_pad = g3.shape[0]
    nblk = rows_pad // (rb * k)

    def body(g_ref, wfc_r, bfc_r, out_ref):
        mp = jnp.max(g_ref[...].reshape(rb, k, 32), axis=1)
        out_ref[...] = _dot(mp.astype(jnp.bfloat16), wfc_r[...]) + bfc_r[...]

    return pl.pallas_call(
        body,
        grid=(nblk,),
        in_specs=[pl.BlockSpec((rb * k, 32), lambda i: (i, 0)),
                  _wspec(wfcp.shape), _wspec(bfcp.shape)],
        out_specs=[pl.BlockSpec((rb, 16), lambda i: (i, 0))],
        out_shape=[jax.ShapeDtypeStruct((rows_pad // k, 16), jnp.float32)],
    )(g3, wfcp, bfcp)[0]


def _stage_i(g4, rows, rb):
    nblk = rows // rb

    def body(g_ref, out_ref):
        out_ref[...] = g_ref[:, 0:13]

    return pl.pallas_call(
        body,
        grid=(nblk,),
        in_specs=[pl.BlockSpec((rb, 16), lambda i: (i, 0))],
        out_specs=[pl.BlockSpec((rb, 13), lambda i: (i, 0))],
        out_shape=[jax.ShapeDtypeStruct((rows, 13), jnp.float32)],
    )(g4)[0]


def kernel(xyz, features, neigh_idx, sub_idx, interp_idx, W_mlp1, b_mlp1,
           W_lfa1, b_lfa1, W_att1_fc, b_att1_fc, W_att1_mlp, b_att1_mlp,
           W_lfa2, b_lfa2, W_att2_fc, b_att2_fc, W_att2_mlp, b_att2_mlp,
           W_mlp2, b_mlp2, W_sc, b_sc, W_fc, b_fc):
    B, N, K = neigh_idx.shape
    Ns = sub_idx.shape[1]
    BN = B * N
    RB = 2048
    BN_PAD = 200704
    f32 = jnp.float32
    bf16 = jnp.bfloat16
    offs_n = (jnp.arange(B, dtype=jnp.int32) * N)[:, None, None]
    offs_s = (jnp.arange(B, dtype=jnp.int32) * Ns)[:, None]
    eyeK = jnp.eye(K, dtype=f32)
    onesK = jnp.ones((K, K), f32)

    wc = W_lfa1[1:4] + W_lfa1[4:7]
    p2 = jnp.zeros((16, 16), f32).at[0:3, 8:16].set(wc)
    p2 = jnp.tile(p2, (1, K))
    blk1 = (jnp.zeros((16, 16), f32)
            .at[0:3, 8:16].set(W_lfa1[7:10] - W_lfa1[1:4])
            .at[3:11, 0:8].set(W_mlp1))
    bd1 = jnp.kron(eyeK, blk1)
    bd2 = jnp.kron(eyeK, W_att1_fc)
    bd3 = jnp.kron(eyeK, jnp.zeros((16, 8), f32).at[8:16].set(W_lfa2))
    gk_ = jnp.kron(onesK, jnp.eye(16, dtype=f32))
    g3 = jnp.kron(eyeK, jnp.ones((16, 16), f32))
    m1 = jnp.zeros((256, 8), f32).at[0:16].set(W_att1_mlp)
    wscp = jnp.zeros((16, 32), f32).at[3:11].set(W_sc)
    w0l = jnp.tile(jnp.zeros((16,), f32).at[8:16].set(W_lfa1[0]), K)[None]
    b1l = jnp.tile(jnp.concatenate([b_mlp1, b_lfa1]), K)[None]
    bfc1l = jnp.tile(b_att1_fc, K)[None]
    b3l = jnp.tile(b_lfa2, K)[None]
    mask3 = ((jnp.arange(256) % 16) < 3).astype(f32)[None]
    cast = lambda a: a.astype(bf16)
    consts_c = (cast(p2), cast(bd1), cast(bd2), cast(bd3), cast(gk_),
                cast(g3), cast(m1), cast(wscp), w0l, b1l, bfc1l, b3l,
                b_att1_mlp[None], b_sc[None], mask3)

    e1 = jnp.kron(eyeK, jnp.concatenate(
        [jnp.eye(8, dtype=f32), jnp.zeros((8, 8), f32)], axis=1))
    e2 = jnp.kron(eyeK, jnp.concatenate(
        [jnp.zeros((8, 8), f32), jnp.eye(8, dtype=f32)], axis=1))
    bd2e = jnp.kron(eyeK, W_att2_fc)
    m2 = jnp.zeros((256, 16), f32).at[0:16].set(W_att2_mlp)
    bfc2l = jnp.tile(b_att2_fc, K)[None]
    consts_e = (cast(e1), cast(e2), cast(bd2e), cast(gk_), cast(m2),
                cast(W_mlp2), bfc2l, b_att2_mlp[None], b_mlp2[None])

    wfcp = jnp.zeros((32, 16), f32).at[:, 0:13].set(W_fc)
    bfcp = jnp.zeros((1, 16), f32).at[0, 0:13].set(b_fc)

    table1 = jnp.concatenate(
        [xyz, features, jnp.zeros((B, N, 5), f32)], axis=-1
    ).reshape(BN, 16)

    gidx1 = _pad_reshape_idx((neigh_idx + offs_n).reshape(-1), 56, 14, BN)
    g1 = _sc_gather(table1, gidx1, 16)

    f_pc_agg, f_xyz2, sc_v = _stage_c(
        g1.reshape(BN_PAD, 256), table1, consts_c, RB, BN_PAD)

    g2 = _sc_gather(f_pc_agg, gidx1, 8)

    f_enc = _stage_e(g2.reshape(BN_PAD, 128), f_xyz2, sc_v, consts_e, RB)

    gidx3 = _pad_reshape_idx((sub_idx + offs_n).reshape(-1), 14, 14, BN)
    g3g = _sc_gather(f_enc, gidx3, 32)
    ls = _stage_g(g3g, cast(wfcp), bfcp, 1024, K)

    gidx4 = _pad_reshape_idx((interp_idx[:, :, 0] + offs_s).reshape(-1), 4, 14, B * Ns)
    g4 = _sc_gather(ls, gidx4, 16)

    logits = _stage_i(g4, BN, 2000)
    return logits.reshape(B, N, 13)


# speedup vs baseline: 1.2297x; 1.0142x over previous
import functools

import jax
import jax.numpy as jnp
from jax import lax
from jax.experimental import pallas as pl
from jax.experimental.pallas import tpu as pltpu
from jax.experimental.pallas import tpu_sc as plsc

_NC = 2
_NS = 16
_NW = _NC * _NS


def _lrelu(x):
    return jnp.where(x >= 0, x, 0.2 * x)


def _dot(a, b):
    return jnp.dot(a, b, preferred_element_type=jnp.float32)


def _sc_gather(table, gidx4d, d, out_dtype=jnp.float32):
    nw, outer, inner, lanes = gidx4d.shape
    ch = inner * lanes
    out_rows = nw * outer * ch
    mesh = plsc.VectorSubcoreMesh(core_axis_name="c", subcore_axis_name="s")

    @functools.partial(
        pl.kernel,
        out_type=jax.ShapeDtypeStruct((out_rows, d), out_dtype),
        mesh=mesh,
        scratch_types=[
            pltpu.VMEM((inner, lanes), jnp.int32),
            pltpu.VMEM((ch, d), out_dtype),
            pltpu.SemaphoreType.DMA,
        ],
        compiler_params=pltpu.CompilerParams(use_tc_tiling_on_sc=False),
    )
    def gk(table_hbm, gidx_hbm, out_hbm, idx_v, rows_v, sem):
        wid = lax.axis_index("s") * _NC + lax.axis_index("c")

        def body(t, carry):
            pltpu.sync_copy(gidx_hbm.at[wid, t], idx_v)
            cps = []
            for j in range(inner):
                cps.append(
                    pltpu.async_copy(
                        table_hbm.at[idx_v.at[j]],
                        rows_v.at[pl.ds(j * lanes, lanes)],
                        sem,
                    )
                )
            for cp in cps:
                cp.wait()
            base = (wid * outer + t) * ch
            pltpu.sync_copy(rows_v, out_hbm.at[pl.ds(base, ch)])
            return carry

        lax.fori_loop(0, outer, body, 0)

    return gk(table, gidx4d)


def _pad_reshape_idx(gidx_flat, outer, inner, table_rows):
    total = _NW * outer * inner * 128
    pad = total - gidx_flat.shape[0]
    filler = jnp.arange(pad, dtype=jnp.int32) % table_rows
    gp = jnp.concatenate([gidx_flat, filler])
    return gp.reshape(_NW, outer, inner, 128)


def _wspec(shp):
    return pl.BlockSpec(shp, lambda i: tuple(0 for _ in shp))


def _stage_c(g1r, tbl, consts, rb, bn_pad):
    nblk = bn_pad // rb
    bf16 = jnp.bfloat16

    def body(g1_ref, tbl_ref, p2_r, bd1_r, bd2_r, bd3_r, g_r, g3_r,
             m1_r, wscp_r, w0l_r, b1l_r, bfc1l_r, b3l_r, bml1_r, bsc_r,
             mask3_r, agg_ref, fx2_ref, sc_ref):
        x = g1_ref[...]
        t = tbl_ref[...]
        tb = t.astype(bf16)
        center = jnp.tile(t, (1, 16))
        relm = (center - x) * mask3_r[...]
        d2 = relm * relm
        dist2 = _dot(d2.astype(bf16), g3_r[...])
        dist = jnp.sqrt(dist2 + 1e-12)
        pre = (_dot(x.astype(bf16), bd1_r[...]) + dist * w0l_r[...]
               + _dot(tb, p2_r[...]) + b1l_r[...])
        fcat = _lrelu(pre)
        fcb = fcat.astype(bf16)
        att = _dot(fcb, bd2_r[...]) + bfc1l_r[...]
        m = att
        for sh in (16, 32, 64, 128):
            m = jnp.maximum(m, pltpu.roll(m, sh, 1))
        e = jnp.exp(att - m)
        den = _dot(e.astype(bf16), g_r[...])
        num = _dot((e * fcat).astype(bf16), g_r[...])
        aggf = num * (1.0 / den)
        agg_ref[...] = _lrelu(_dot(aggf.astype(bf16), m1_r[...]) + bml1_r[...])
        fx2_ref[...] = _lrelu(_dot(fcb, bd3_r[...]) + b3l_r[...]).astype(bf16)
        sc_ref[...] = (_dot(tb, wscp_r[...]) + bsc_r[...]).astype(bf16)

    return pl.pallas_call(
        body,
        grid=(nblk,),
        in_specs=[
            pl.BlockSpec((rb, 256), lambda i: (i, 0)),
            pl.BlockSpec((rb, 16), lambda i: (i, 0)),
        ] + [_wspec(c.shape) for c in consts],
        out_specs=[
            pl.BlockSpec((rb, 8), lambda i: (i, 0)),
            pl.BlockSpec((rb, 128), lambda i: (i, 0)),
            pl.BlockSpec((rb, 32), lambda i: (i, 0)),
        ],
        out_shape=[
            jax.ShapeDtypeStruct((bn_pad, 8), jnp.float32),
            jax.ShapeDtypeStruct((bn_pad, 128), jnp.bfloat16),
            jax.ShapeDtypeStruct((bn_pad, 32), jnp.bfloat16),
        ],
    )(g1r, tbl, *consts)


def _stage_e(g2r, fx2r, scv, consts, rb):
    bn_pad = scv.shape[0]
    nblk = bn_pad // rb
    bf16 = jnp.bfloat16

    def body(g2_ref, fx2_ref, sc_ref, e1_r, e2_r, bd2e_r, g_r, m2_r, wm2_r,
             bfc2l_r, bml2_r, bm2_r, enc_ref):
        fcat = (_dot(g2_ref[...].astype(bf16), e1_r[...])
                + _dot(fx2_ref[...], e2_r[...]))
        fcb = fcat.astype(bf16)
        att = _dot(fcb, bd2e_r[...]) + bfc2l_r[...]
        m = att
        for sh in (16, 32, 64, 128):
            m = jnp.maximum(m, pltpu.roll(m, sh, 1))
        e = jnp.exp(att - m)
        den = _dot(e.astype(bf16), g_r[...])
        num = _dot((e * fcat).astype(bf16), g_r[...])
        aggf = num * (1.0 / den)
        f_lfa = _lrelu(_dot(aggf.astype(bf16), m2_r[...]) + bml2_r[...])
        f_main = _dot(f_lfa.astype(bf16), wm2_r[...]) + bm2_r[...]
        enc_ref[...] = _lrelu(f_main + sc_ref[...].astype(jnp.float32))

    return pl.pallas_call(
        body,
        grid=(nblk,),
        in_specs=[
            pl.BlockSpec((rb, 128), lambda i: (i, 0)),
            pl.BlockSpec((rb, 128), lambda i: (i, 0)),
            pl.BlockSpec((rb, 32), lambda i: (i, 0)),
        ] + [_wspec(c.shape) for c in consts],
        out_specs=[pl.BlockSpec((rb, 32), lambda i: (i, 0))],
        out_shape=[jax.ShapeDtypeStruct((bn_pad, 32), jnp.float32)],
    )(g2r, fx2r, scv, *consts)[0]


def _stage_g(g3, wfcp, bfcp, rb, k):
    rows_pad = g3.shape[0]
    nblk = rows_pad // (rb * k)

    def body(g_ref, wfc_r, bfc_r, out_ref):
        mp = jnp.max(g_ref[...].reshape(rb, k, 32), axis=1)
        out_ref[...] = _dot(mp.astype(jnp.bfloat16), wfc_r[...]) + bfc_r[...]

    return pl.pallas_call(
        body,
        grid=(nblk,),
        in_specs=[pl.BlockSpec((rb * k, 32), lambda i: (i, 0)),
                  _wspec(wfcp.shape), _wspec(bfcp.shape)],
        out_specs=[pl.BlockSpec((rb, 16), lambda i: (i, 0))],
        out_shape=[jax.ShapeDtypeStruct((rows_pad // k, 16), jnp.float32)],
    )(g3, wfcp, bfcp)[0]


def _stage_i(g4, rows, rb):
    nblk = rows // rb

    def body(g_ref, out_ref):
        out_ref[...] = g_ref[:, 0:13]

    return pl.pallas_call(
        body,
        grid=(nblk,),
        in_specs=[pl.BlockSpec((rb, 16), lambda i: (i, 0))],
        out_specs=[pl.BlockSpec((rb, 13), lambda i: (i, 0))],
        out_shape=[jax.ShapeDtypeStruct((rows, 13), jnp.float32)],
    )(g4)[0]


def kernel(xyz, features, neigh_idx, sub_idx, interp_idx, W_mlp1, b_mlp1,
           W_lfa1, b_lfa1, W_att1_fc, b_att1_fc, W_att1_mlp, b_att1_mlp,
           W_lfa2, b_lfa2, W_att2_fc, b_att2_fc, W_att2_mlp, b_att2_mlp,
           W_mlp2, b_mlp2, W_sc, b_sc, W_fc, b_fc):
    B, N, K = neigh_idx.shape
    Ns = sub_idx.shape[1]
    BN = B * N
    RB = 4096
    BN_PAD = 200704
    f32 = jnp.float32
    bf16 = jnp.bfloat16
    offs_n = (jnp.arange(B, dtype=jnp.int32) * N)[:, None, None]
    offs_s = (jnp.arange(B, dtype=jnp.int32) * Ns)[:, None]
    eyeK = jnp.eye(K, dtype=f32)
    onesK = jnp.ones((K, K), f32)

    wc = W_lfa1[1:4] + W_lfa1[4:7]
    p2 = jnp.zeros((16, 16), f32).at[0:3, 8:16].set(wc)
    p2 = jnp.tile(p2, (1, K))
    blk1 = (jnp.zeros((16, 16), f32)
            .at[0:3, 8:16].set(W_lfa1[7:10] - W_lfa1[1:4])
            .at[3:11, 0:8].set(W_mlp1))
    bd1 = jnp.kron(eyeK, blk1)
    bd2 = jnp.kron(eyeK, W_att1_fc)
    bd3 = jnp.kron(eyeK, jnp.zeros((16, 8), f32).at[8:16].set(W_lfa2))
    gk_ = jnp.kron(onesK, jnp.eye(16, dtype=f32))
    g3 = jnp.kron(eyeK, jnp.ones((16, 16), f32))
    m1 = jnp.zeros((256, 8), f32).at[0:16].set(W_att1_mlp)
    wscp = jnp.zeros((16, 32), f32).at[3:11].set(W_sc)
    w0l = jnp.tile(jnp.zeros((16,), f32).at[8:16].set(W_lfa1[0]), K)[None]
    b1l = jnp.tile(jnp.concatenate([b_mlp1, b_lfa1]), K)[None]
    bfc1l = jnp.tile(b_att1_fc, K)[None]
    b3l = jnp.tile(b_lfa2, K)[None]
    mask3 = ((jnp.arange(256) % 16) < 3).astype(f32)[None]
    cast = lambda a: a.astype(bf16)
    consts_c = (cast(p2), cast(bd1), cast(bd2), cast(bd3), cast(gk_),
                cast(g3), cast(m1), cast(wscp), w0l, b1l, bfc1l, b3l,
                b_att1_mlp[None], b_sc[None], mask3)

    e1 = jnp.kron(eyeK, jnp.concatenate(
        [jnp.eye(8, dtype=f32), jnp.zeros((8, 8), f32)], axis=1))
    e2 = jnp.kron(eyeK, jnp.concatenate(
        [jnp.zeros((8, 8), f32), jnp.eye(8, dtype=f32)], axis=1))
    bd2e = jnp.kron(eyeK, W_att2_fc)
    m2 = jnp.zeros((256, 16), f32).at[0:16].set(W_att2_mlp)
    bfc2l = jnp.tile(b_att2_fc, K)[None]
    consts_e = (cast(e1), cast(e2), cast(bd2e), cast(gk_), cast(m2),
                cast(W_mlp2), bfc2l, b_att2_mlp[None], b_mlp2[None])

    wfcp = jnp.zeros((32, 16), f32).at[:, 0:13].set(W_fc)
    bfcp = jnp.zeros((1, 16), f32).at[0, 0:13].set(b_fc)

    table1 = jnp.concatenate(
        [xyz, features, jnp.zeros((B, N, 5), f32)], axis=-1
    ).reshape(BN, 16)

    gidx1 = _pad_reshape_idx((neigh_idx + offs_n).reshape(-1), 56, 14, BN)
    g1 = _sc_gather(table1, gidx1, 16)

    f_pc_agg, f_xyz2, sc_v = _stage_c(
        g1.reshape(BN_PAD, 256), table1, consts_c, RB, BN_PAD)

    g2 = _sc_gather(f_pc_agg, gidx1, 8)

    f_enc = _stage_e(g2.reshape(BN_PAD, 128), f_xyz2, sc_v, consts_e, RB)

    gidx3 = _pad_reshape_idx((sub_idx + offs_n).reshape(-1), 14, 14, BN)
    g3g = _sc_gather(f_enc, gidx3, 32)
    ls = _stage_g(g3g, cast(wfcp), bfcp, 1024, K)

    gidx4 = _pad_reshape_idx((interp_idx[:, :, 0] + offs_s).reshape(-1), 4, 14, B * Ns)
    g4 = _sc_gather(ls, gidx4, 16)

    logits = _stage_i(g4, BN, 2000)
    return logits.reshape(B, N, 13)
